# Initial kernel scaffold; baseline (speedup 1.0000x reference)
import jax
import jax.numpy as jnp
from jax.experimental import pallas as pl


def kernel(current_node_ids, network_features, edge_index, edge_attr, params):
    raise NotImplementedError("write your pallas kernel here")


# speedup vs baseline: 27.8327x; 27.8327x over previous
import functools

import jax
import jax.numpy as jnp
from jax import lax
from jax.experimental import pallas as pl
from jax.experimental.pallas import tpu as pltpu
from jax.experimental.pallas import tpu_sc as plsc

N = 10000
E = 320000
HID = 128
EDIM = 16
NF = 16
B = 1024

G = 128
NW = 32
NR = 10240
RPT = NR // 16
PAD_ROW = NR - 1

CH_A = 81
EN_PAD = NW * CH_A * G
PT_A = CH_A * G

CH_0 = 79
E_PAD = NW * CH_0 * G
PT_0 = CH_0 * G

_f32 = jnp.float32
_i32 = jnp.int32

_MESH = plsc.VectorSubcoreMesh(core_axis_name="c", subcore_axis_name="s")
_SC_PARAMS = pltpu.CompilerParams(use_tc_tiling_on_sc=False,
                                  needs_layout_passes=False)


def _tile_ids():
    cid = lax.axis_index("c")
    sid = lax.axis_index("s")
    return cid, sid, cid * 16 + sid


def _fill_rows(ref, nrows, ncols, val):
    def body(i, _):
        for h in range(ncols // 16):
            ref[i, pl.ds(16 * h, 16)] = jnp.full((16,), val, _f32)
        return 0
    lax.fori_loop(0, nrows, body, 0)


@functools.partial(
    pl.kernel,
    out_type=(
        jax.ShapeDtypeStruct((2, NR, EDIM), _f32),
        jax.ShapeDtypeStruct((2, NR, EDIM), _f32),
    ),
    mesh=_MESH,
    compiler_params=_SC_PARAMS,
    scratch_types=(
        pltpu.VMEM((G,), _i32),
        pltpu.VMEM((G, EDIM), _f32),
        pltpu.VMEM((G, EDIM), _f32),
        pltpu.VMEM_SHARED((NR, EDIM), _f32),
        pltpu.VMEM_SHARED((NR, EDIM), _f32),
    ),
)
def _sc_loopattr(d_hbm, ea_hbm, easum_out, cnt_out,
                 didx_v, ear_v, ones_v, accea_s, acccnt_s):
    cid, sid, wid = _tile_ids()
    _fill_rows(ear_v, G, EDIM, 0.0)
    _fill_rows(ones_v, G, EDIM, 1.0)
    for t in range(RPT // G):
        pltpu.sync_copy(ear_v, accea_s.at[pl.ds(sid * RPT + t * G, G)])
        pltpu.sync_copy(ear_v, acccnt_s.at[pl.ds(sid * RPT + t * G, G)])
    plsc.subcore_barrier()

    def chunk(k, _):
        base = wid * PT_0 + k * G
        pltpu.sync_copy(d_hbm.at[pl.ds(base, G)], didx_v)
        pltpu.sync_copy(ea_hbm.at[pl.ds(base, G)], ear_v)
        pltpu.sync_copy(ear_v, accea_s.at[didx_v], add=True)
        pltpu.sync_copy(ones_v, acccnt_s.at[didx_v], add=True)
        return 0

    lax.fori_loop(0, CH_0, chunk, 0)
    plsc.subcore_barrier()
    for t in range(RPT // G):
        r = sid * RPT + t * G
        pltpu.sync_copy(accea_s.at[pl.ds(r, G)], ear_v)
        pltpu.sync_copy(ear_v, easum_out.at[cid, pl.ds(r, G)])
        pltpu.sync_copy(acccnt_s.at[pl.ds(r, G)], ones_v)
        pltpu.sync_copy(ones_v, cnt_out.at[cid, pl.ds(r, G)])


def _make_sc_edge(nheads):
    @functools.partial(
        pl.kernel,
        out_type=(
            jax.ShapeDtypeStruct((2, NR, HID), _f32),
            jax.ShapeDtypeStruct((2, NR, EDIM), _f32),
        ),
        mesh=_MESH,
        compiler_params=_SC_PARAMS,
        scratch_types=(
            pltpu.VMEM((G,), _i32),
            pltpu.VMEM((G,), _i32),
            pltpu.VMEM((G, EDIM), _f32),
            pltpu.VMEM((G, EDIM), _f32),
            pltpu.VMEM((G, EDIM), _f32),
            pltpu.VMEM((G, EDIM), _f32),
            pltpu.VMEM((G, HID), _f32),
            pltpu.VMEM_SHARED((NR, HID), _f32),
            pltpu.VMEM_SHARED((NR, EDIM), _f32),
            pltpu.SemaphoreType.DMA,
            pltpu.SemaphoreType.DMA,
            pltpu.SemaphoreType.DMA,
        ),
    )
    def edge_pass(s2_hbm, d2_hbm, asrc_hbm, adst_hbm, ae_hbm, xw_hbm,
                  out_hbm, den_hbm,
                  sidx_v, didx_v, asr_v, adr_v, aer_v, exr_v, xwr_v,
                  out_s, den_s, sem_a, sem_b, sem_x):
        cid, sid, wid = _tile_ids()
        _fill_rows(xwr_v, G, HID, 0.0)
        _fill_rows(exr_v, G, EDIM, 0.0)
        for t in range(RPT // G):
            pltpu.sync_copy(xwr_v, out_s.at[pl.ds(sid * RPT + t * G, G)])
            pltpu.sync_copy(exr_v, den_s.at[pl.ds(sid * RPT + t * G, G)])
        plsc.subcore_barrier()

        def chunk(k, _):
            base = wid * PT_A + k * G
            pltpu.sync_copy(s2_hbm.at[pl.ds(base, G)], sidx_v)
            pltpu.sync_copy(d2_hbm.at[pl.ds(base, G)], didx_v)
            cx = pltpu.async_copy(xw_hbm.at[sidx_v], xwr_v, sem_x)
            ca = pltpu.async_copy(asrc_hbm.at[sidx_v], asr_v, sem_a)
            cb = pltpu.async_copy(adst_hbm.at[didx_v], adr_v, sem_b)
            pltpu.sync_copy(ae_hbm.at[pl.ds(base, G)], aer_v)
            ca.wait()
            cb.wait()

            def ex_row(i, _):
                a = asr_v[i] + adr_v[i] + aer_v[i]
                a = jnp.where(a > 0.0, a, 0.2 * a)
                exr_v[i] = jnp.exp(a)
                return 0

            lax.fori_loop(0, G, ex_row, 0)
            cx.wait()

            def scale_row(i, _):
                ii = jnp.broadcast_to(i, (16,)).astype(_i32)
                for h in range(HID // 16):
                    hh = h if nheads == 8 else 0
                    m = plsc.load_gather(
                        exr_v, [ii, jnp.full((16,), hh, _i32)])
                    xwr_v[i, pl.ds(16 * h, 16)] = (
                        xwr_v[i, pl.ds(16 * h, 16)] * m)
                return 0

            lax.fori_loop(0, G, scale_row, 0)
            pltpu.sync_copy(xwr_v, out_s.at[didx_v], add=True)
            pltpu.sync_copy(exr_v, den_s.at[didx_v], add=True)
            return 0

        lax.fori_loop(0, CH_A, chunk, 0)
        plsc.subcore_barrier()
        for t in range(RPT // G):
            r = sid * RPT + t * G
            pltpu.sync_copy(out_s.at[pl.ds(r, G)], xwr_v)
            pltpu.sync_copy(xwr_v, out_hbm.at[cid, pl.ds(r, G)])
            pltpu.sync_copy(den_s.at[pl.ds(r, G)], exr_v)
            pltpu.sync_copy(exr_v, den_hbm.at[cid, pl.ds(r, G)])

    return edge_pass


_sc_edge_h8 = _make_sc_edge(8)
_sc_edge_h1 = _make_sc_edge(1)


@functools.partial(
    pl.kernel,
    out_type=jax.ShapeDtypeStruct((B, HID), _f32),
    mesh=_MESH,
    compiler_params=_SC_PARAMS,
    scratch_types=(
        pltpu.VMEM((B // NW,), _i32),
        pltpu.VMEM((B // NW, HID), _f32),
        pltpu.SemaphoreType.DMA,
    ),
)
def _sc_gather_rows(ids_hbm, x_hbm, out_hbm, idx_v, rows_v, sem):
    _, _, wid = _tile_ids()
    base = wid * (B // NW)
    pltpu.sync_copy(ids_hbm.at[pl.ds(base, B // NW)], idx_v)
    pltpu.async_copy(x_hbm.at[idx_v], rows_v, sem).wait()
    pltpu.sync_copy(rows_v, out_hbm.at[pl.ds(base, B // NW)])


_BN_INV = 0.9999950000374997


def _head_sel(chan):
    jr = lax.broadcasted_iota(_i32, (HID, 16), 0)
    hc = lax.broadcasted_iota(_i32, (HID, 16), 1)
    return (jr // chan == hc).astype(_f32)


def _head_expand(chan):
    hc = lax.broadcasted_iota(_i32, (16, HID), 0)
    jr = lax.broadcasted_iota(_i32, (16, HID), 1)
    return (jr // chan == hc).astype(_f32)


def _proj_body(x, w_ref, asf_ref, adf_ref, chan, xw_ref, as_ref, ad_ref):
    xw = jnp.dot(x, w_ref[...], preferred_element_type=_f32)
    sel = _head_sel(chan)
    as_ref[...] = jnp.dot(xw * asf_ref[...], sel, preferred_element_type=_f32)
    ad_ref[...] = jnp.dot(xw * adf_ref[...], sel, preferred_element_type=_f32)
    xw_ref[...] = xw


def _tc_proj_first(chan):
    def body(x_ref, w_ref, asf_ref, adf_ref, xw_ref, as_ref, ad_ref):
        _proj_body(x_ref[...], w_ref, asf_ref, adf_ref, chan,
                   xw_ref, as_ref, ad_ref)
    return body


def _tc_proj_next(chan_prev, chan, relu):
    def body(p0_ref, p1_ref, d0_ref, d1_ref, b_ref, w_ref, asf_ref, adf_ref,
             xw_ref, as_ref, ad_ref):
        dinv = 1.0 / (d0_ref[...] + d1_ref[...] + 1e-16)
        scale = jnp.dot(dinv, _head_expand(chan_prev),
                        preferred_element_type=_f32)
        x = (p0_ref[...] + p1_ref[...]) * scale + b_ref[...]
        if relu:
            x = jnp.maximum(x, 0.0)
        _proj_body(x, w_ref, asf_ref, adf_ref, chan, xw_ref, as_ref, ad_ref)
    return body


_PROJ_OUT = (
    jax.ShapeDtypeStruct((NR, HID), _f32),
    jax.ShapeDtypeStruct((NR, 16), _f32),
    jax.ShapeDtypeStruct((NR, 16), _f32),
)
_PROJ_OUT_SPECS = (
    pl.BlockSpec((1024, HID), lambda i: (i, 0)),
    pl.BlockSpec((1024, 16), lambda i: (i, 0)),
    pl.BlockSpec((1024, 16), lambda i: (i, 0)),
)
_W_SPEC = pl.BlockSpec((HID, HID), lambda i: (0, 0))
_ROW_SPEC = pl.BlockSpec((1, HID), lambda i: (0, 0))
_X_SPEC = pl.BlockSpec((1024, HID), lambda i: (i, 0))
_D_SPEC = pl.BlockSpec((1024, 16), lambda i: (i, 0))


def _run_proj_first(x_pad, w, asf, adf, chan):
    return pl.pallas_call(
        _tc_proj_first(chan),
        grid=(NR // 1024,),
        in_specs=[_X_SPEC, _W_SPEC, _ROW_SPEC, _ROW_SPEC],
        out_shape=_PROJ_OUT,
        out_specs=_PROJ_OUT_SPECS,
    )(x_pad, w, asf, adf)


def _run_proj_next(out_p, den_p, bias, w, asf, adf, chan_prev, chan):
    return pl.pallas_call(
        _tc_proj_next(chan_prev, chan, True),
        grid=(NR // 1024,),
        in_specs=[_X_SPEC, _X_SPEC, _D_SPEC, _D_SPEC, _ROW_SPEC,
                  _W_SPEC, _ROW_SPEC, _ROW_SPEC],
        out_shape=_PROJ_OUT,
        out_specs=_PROJ_OUT_SPECS,
    )(out_p[0], out_p[1], den_p[0], den_p[1], bias, w, asf, adf)


def _loop_attr_body(ea0_ref, ea1_ref, c0_ref, c1_ref, out_ref):
    cnt = jnp.maximum(c0_ref[...] + c1_ref[...], 1.0)
    out_ref[...] = (ea0_ref[...] + ea1_ref[...]) / cnt


def _edge_attn_body(ea_ref, we1_ref, af1_ref, we2_ref, af2_ref,
                    we3_ref, af3_ref, o1_ref, o2_ref, o3_ref):
    ea = ea_ref[...]
    s16 = _head_sel(16)
    s1 = _head_sel(HID)
    for we, af, sel, out in ((we1_ref, af1_ref, s16, o1_ref),
                             (we2_ref, af2_ref, s16, o2_ref),
                             (we3_ref, af3_ref, s1, o3_ref)):
        m = jnp.dot(we[...] * af[...], sel, preferred_element_type=_f32)
        out[...] = jnp.dot(ea, m, preferred_element_type=_f32)


def _final_x_body(p0_ref, p1_ref, d0_ref, d1_ref, b_ref, out_ref):
    dinv = 1.0 / (d0_ref[...] + d1_ref[...] + 1e-16)
    scale = jnp.dot(dinv, _head_expand(HID), preferred_element_type=_f32)
    out_ref[...] = (p0_ref[...] + p1_ref[...]) * scale + b_ref[...]


def _bn(x, g, b):
    return x * (g * _BN_INV) + b


def _fuse_body(nf_ref, w1_ref, b1_ref, g1_ref, be1_ref,
               w2_ref, b2_ref, g2_ref, be2_ref,
               ne_ref, fw_ref, fb_ref, fg_ref, fbe_ref, out_ref):
    h = jnp.dot(nf_ref[...], w1_ref[...], preferred_element_type=_f32)
    h = jnp.maximum(_bn(h + b1_ref[...], g1_ref[...], be1_ref[...]), 0.0)
    h = jnp.dot(h, w2_ref[...], preferred_element_type=_f32)
    h = jnp.maximum(_bn(h + b2_ref[...], g2_ref[...], be2_ref[...]), 0.0)
    f = (jnp.dot(ne_ref[...], fw_ref[0:HID, :], preferred_element_type=_f32)
         + jnp.dot(h, fw_ref[HID:2 * HID, :], preferred_element_type=_f32))
    f = jnp.maximum(_bn(f + fb_ref[...], fg_ref[...], fbe_ref[...]), 0.0)
    out_ref[...] = f


def _cls_body(f_ref, w_ref, b_ref, out_ref):
    out_ref[...] = (jnp.dot(f_ref[...], w_ref[...],
                            preferred_element_type=_f32) + b_ref[...])


def kernel(current_node_ids, network_features, edge_index, edge_attr, params):
    src = edge_index[0]
    dst = edge_index[1]
    loop_ids = jnp.arange(N, dtype=_i32)
    pad_a = jnp.full((EN_PAD - E - N,), PAD_ROW, _i32)
    s2 = jnp.concatenate([src, loop_ids, pad_a])
    d2 = jnp.concatenate([dst, loop_ids, pad_a])
    d0 = jnp.concatenate([dst, jnp.full((E_PAD - E,), PAD_ROW, _i32)])
    ea_pad = jnp.pad(edge_attr, ((0, E_PAD - E), (0, 0)))

    easum_p, cnt_p = _sc_loopattr(d0, ea_pad)
    loop_attr = pl.pallas_call(
        _loop_attr_body,
        grid=(NR // 1024,),
        in_specs=[_D_SPEC] * 4,
        out_shape=jax.ShapeDtypeStruct((NR, EDIM), _f32),
        out_specs=_D_SPEC,
    )(easum_p[0], easum_p[1], cnt_p[0], cnt_p[1])

    ea2 = jnp.concatenate(
        [edge_attr, loop_attr[:N], jnp.zeros((EN_PAD - E - N, EDIM), _f32)])

    g1p, g2p, g3p = params['gat1'], params['gat2'], params['gat3']
    af = [p['att_e'].reshape(1, HID) for p in (g1p, g2p, g3p)]
    ae1, ae2, ae3 = pl.pallas_call(
        _edge_attn_body,
        grid=(EN_PAD // 2048,),
        in_specs=[
            pl.BlockSpec((2048, EDIM), lambda i: (i, 0)),
            pl.BlockSpec((EDIM, HID), lambda i: (0, 0)),
            _ROW_SPEC,
            pl.BlockSpec((EDIM, HID), lambda i: (0, 0)),
            _ROW_SPEC,
            pl.BlockSpec((EDIM, HID), lambda i: (0, 0)),
            _ROW_SPEC,
        ],
        out_shape=tuple(
            jax.ShapeDtypeStruct((EN_PAD, EDIM), _f32) for _ in range(3)),
        out_specs=tuple(
            pl.BlockSpec((2048, EDIM), lambda i: (i, 0)) for _ in range(3)),
    )(ea2, g1p['W_e'], af[0], g2p['W_e'], af[1], g3p['W_e'], af[2])

    emb_pad = jnp.pad(params['emb'], ((0, NR - N), (0, 0)))

    xw, asrc, adst = _run_proj_first(
        emb_pad, g1p['W'], g1p['att_src'].reshape(1, HID),
        g1p['att_dst'].reshape(1, HID), 16)
    out_p, den_p = _sc_edge_h8(s2, d2, asrc, adst, ae1, xw)

    xw, asrc, adst = _run_proj_next(
        out_p, den_p, g1p['b'].reshape(1, HID), g2p['W'],
        g2p['att_src'].reshape(1, HID), g2p['att_dst'].reshape(1, HID),
        16, 16)
    out_p, den_p = _sc_edge_h8(s2, d2, asrc, adst, ae2, xw)

    xw, asrc, adst = _run_proj_next(
        out_p, den_p, g2p['b'].reshape(1, HID), g3p['W'],
        g3p['att_src'].reshape(1, HID), g3p['att_dst'].reshape(1, HID),
        16, HID)
    out_p, den_p = _sc_edge_h1(s2, d2, asrc, adst, ae3, xw)

    x3 = pl.pallas_call(
        _final_x_body,
        grid=(NR // 1024,),
        in_specs=[_X_SPEC, _X_SPEC, _D_SPEC, _D_SPEC, _ROW_SPEC],
        out_shape=jax.ShapeDtypeStruct((NR, HID), _f32),
        out_specs=_X_SPEC,
    )(out_p[0], out_p[1], den_p[0], den_p[1], g3p['b'].reshape(1, HID))

    node_emb = _sc_gather_rows(current_node_ids, x3)

    fused = pl.pallas_call(
        _fuse_body,
        out_shape=jax.ShapeDtypeStruct((B, HID), _f32),
    )(network_features,
      params['ne_W1'], params['ne_b1'].reshape(1, HID),
      params['ne_g1'].reshape(1, HID), params['ne_be1'].reshape(1, HID),
      params['ne_W2'], params['ne_b2'].reshape(1, HID),
      params['ne_g2'].reshape(1, HID), params['ne_be2'].reshape(1, HID),
      node_emb, params['fus_W'], params['fus_b'].reshape(1, HID),
      params['fus_g'].reshape(1, HID), params['fus_be'].reshape(1, HID))

    cls_w = jnp.pad(params['cls_W'], ((0, 0), (0, NR - N)))
    cls_b = jnp.pad(params['cls_b'], (0, NR - N)).reshape(1, NR)
    logits = pl.pallas_call(
        _cls_body,
        grid=(NR // 1024,),
        in_specs=[
            pl.BlockSpec((B, HID), lambda i: (0, 0)),
            pl.BlockSpec((HID, 1024), lambda i: (0, i)),
            pl.BlockSpec((1, 1024), lambda i: (0, i)),
        ],
        out_shape=jax.ShapeDtypeStruct((B, NR), _f32),
        out_specs=pl.BlockSpec((B, 1024), lambda i: (0, i)),
    )(fused, cls_w, cls_b)
    return logits[:, :N]


# speedup vs baseline: 30.2959x; 1.0885x over previous
import functools

import jax
import jax.numpy as jnp
from jax import lax
from jax.experimental import pallas as pl
from jax.experimental.pallas import tpu as pltpu
from jax.experimental.pallas import tpu_sc as plsc

N = 10000
E = 320000
HID = 128
EDIM = 16
NF = 16
B = 1024

G = 128
NW = 32
NR = 10240
RPT = NR // 16
PAD_ROW = NR - 1

CH_A = 82
EN_PAD = NW * CH_A * G
PT_A = CH_A * G
XWE = HID + 16

CH_0 = 79
E_PAD = NW * CH_0 * G
PT_0 = CH_0 * G

_f32 = jnp.float32
_i32 = jnp.int32

_MESH = plsc.VectorSubcoreMesh(core_axis_name="c", subcore_axis_name="s")
_SC_PARAMS = pltpu.CompilerParams(use_tc_tiling_on_sc=False,
                                  needs_layout_passes=False)


def _tile_ids():
    cid = lax.axis_index("c")
    sid = lax.axis_index("s")
    return cid, sid, cid * 16 + sid


def _fill_rows(ref, nrows, ncols, val):
    def body(i, _):
        for h in range(ncols // 16):
            ref[i, pl.ds(16 * h, 16)] = jnp.full((16,), val, _f32)
        return 0
    lax.fori_loop(0, nrows, body, 0)


@functools.partial(
    pl.kernel,
    out_type=(
        jax.ShapeDtypeStruct((2, NR, EDIM), _f32),
        jax.ShapeDtypeStruct((2, NR, EDIM), _f32),
    ),
    mesh=_MESH,
    compiler_params=_SC_PARAMS,
    scratch_types=(
        pltpu.VMEM((G,), _i32),
        pltpu.VMEM((G, EDIM), _f32),
        pltpu.VMEM((G, EDIM), _f32),
        pltpu.VMEM_SHARED((NR, EDIM), _f32),
        pltpu.VMEM_SHARED((NR, EDIM), _f32),
    ),
)
def _sc_loopattr(d_hbm, ea_hbm, easum_out, cnt_out,
                 didx_v, ear_v, ones_v, accea_s, acccnt_s):
    cid, sid, wid = _tile_ids()
    _fill_rows(ear_v, G, EDIM, 0.0)
    _fill_rows(ones_v, G, EDIM, 1.0)
    for t in range(RPT // G):
        pltpu.sync_copy(ear_v, accea_s.at[pl.ds(sid * RPT + t * G, G)])
        pltpu.sync_copy(ear_v, acccnt_s.at[pl.ds(sid * RPT + t * G, G)])
    plsc.subcore_barrier()

    def chunk(k, _):
        base = wid * PT_0 + k * G
        pltpu.sync_copy(d_hbm.at[pl.ds(base, G)], didx_v)
        pltpu.sync_copy(ea_hbm.at[pl.ds(base, G)], ear_v)
        pltpu.sync_copy(ear_v, accea_s.at[didx_v], add=True)
        pltpu.sync_copy(ones_v, acccnt_s.at[didx_v], add=True)
        return 0

    lax.fori_loop(0, CH_0, chunk, 0)
    plsc.subcore_barrier()
    for t in range(RPT // G):
        r = sid * RPT + t * G
        pltpu.sync_copy(accea_s.at[pl.ds(r, G)], ear_v)
        pltpu.sync_copy(ear_v, easum_out.at[cid, pl.ds(r, G)])
        pltpu.sync_copy(acccnt_s.at[pl.ds(r, G)], ones_v)
        pltpu.sync_copy(ones_v, cnt_out.at[cid, pl.ds(r, G)])


def _make_sc_edge(nheads):
    GE = 64
    CH = PT_A // GE

    @functools.partial(
        pl.kernel,
        out_type=(
            jax.ShapeDtypeStruct((2, NR, HID), _f32),
            jax.ShapeDtypeStruct((2, NR, EDIM), _f32),
        ),
        mesh=_MESH,
        compiler_params=_SC_PARAMS,
        scratch_types=(
            pltpu.VMEM((2, GE), _i32),
            pltpu.VMEM((2, GE), _i32),
            pltpu.VMEM((2, GE, XWE), _f32),
            pltpu.VMEM((2, GE, EDIM), _f32),
            pltpu.VMEM((2, GE, EDIM), _f32),
            pltpu.VMEM((GE, EDIM), _f32),
            pltpu.VMEM((GE, HID), _f32),
            pltpu.VMEM((GE, EDIM), _f32),
            pltpu.VMEM((GE,), _i32),
            pltpu.VMEM_SHARED((NR, HID), _f32),
            pltpu.VMEM_SHARED((NR, EDIM), _f32),
        ) + (pltpu.SemaphoreType.DMA,) * 10,
    )
    def edge_pass(s2_hbm, d2_hbm, adst_hbm, ae_hbm, xwe_hbm,
                  out_hbm, den_hbm,
                  sidx_v, didx_v, xwe_v, adr_v, aer_v, exr_v, xws_v, ste_v,
                  dsc_v, out_s, den_s, *sems):
        cid, sid, wid = _tile_ids()
        s_si = sems[0:2]
        s_di = sems[2:4]
        s_gx = sems[4:6]
        s_ga = sems[6:8]
        s_ge = sems[8:10]

        def fire_idx(k, s):
            base = wid * PT_A + k * GE
            pltpu.async_copy(s2_hbm.at[pl.ds(base, GE)], sidx_v.at[s], s_si[s])
            pltpu.async_copy(d2_hbm.at[pl.ds(base, GE)], didx_v.at[s], s_di[s])

        def wait_idx(s):
            pltpu.make_async_copy(
                s2_hbm.at[pl.ds(0, GE)], sidx_v.at[s], s_si[s]).wait()
            pltpu.make_async_copy(
                d2_hbm.at[pl.ds(0, GE)], didx_v.at[s], s_di[s]).wait()

        def fire_rows(k, s):
            base = wid * PT_A + k * GE
            pltpu.async_copy(xwe_hbm.at[sidx_v.at[s]], xwe_v.at[s], s_gx[s])
            pltpu.async_copy(adst_hbm.at[didx_v.at[s]], adr_v.at[s], s_ga[s])
            pltpu.async_copy(ae_hbm.at[pl.ds(base, GE)], aer_v.at[s], s_ge[s])

        def wait_rows(s):
            pltpu.make_async_copy(
                xwe_hbm.at[sidx_v.at[s]], xwe_v.at[s], s_gx[s]).wait()
            pltpu.make_async_copy(
                adst_hbm.at[didx_v.at[s]], adr_v.at[s], s_ga[s]).wait()
            pltpu.make_async_copy(
                ae_hbm.at[pl.ds(0, GE)], aer_v.at[s], s_ge[s]).wait()

        def compute(s):
            def row(i, _):
                a = (xwe_v[s, i, pl.ds(HID, 16)] + adr_v[s, i]
                     + aer_v[s, i])
                a = jnp.where(a > 0.0, a, 0.2 * a)
                exr_v[i] = jnp.exp(a)
                ii = jnp.broadcast_to(i, (16,)).astype(_i32)
                for h in range(HID // 16):
                    hh = h if nheads == 8 else 0
                    m = plsc.load_gather(
                        exr_v, [ii, jnp.full((16,), hh, _i32)])
                    xws_v[i, pl.ds(16 * h, 16)] = (
                        xwe_v[s, i, pl.ds(16 * h, 16)] * m)
                return 0

            lax.fori_loop(0, GE, row, 0)

        def step(k, s, do_rows_next, do_idx2):
            if do_rows_next:
                wait_idx(1 - s)
            wait_rows(s)
            if do_rows_next:
                fire_rows(k + 1, 1 - s)
            for t in range(GE // 16):
                dsc_v[pl.ds(16 * t, 16)] = didx_v[s, pl.ds(16 * t, 16)]
            if do_idx2:
                fire_idx(k + 2, s)
            compute(s)
            pltpu.sync_copy(xws_v, out_s.at[dsc_v], add=True)
            pltpu.sync_copy(exr_v, den_s.at[dsc_v], add=True)

        _fill_rows(xws_v, GE, HID, 0.0)
        _fill_rows(ste_v, GE, EDIM, 0.0)
        for t in range(RPT // GE):
            pltpu.sync_copy(xws_v, out_s.at[pl.ds(sid * RPT + t * GE, GE)])
            pltpu.sync_copy(ste_v, den_s.at[pl.ds(sid * RPT + t * GE, GE)])
        plsc.subcore_barrier()

        fire_idx(0, 0)
        wait_idx(0)
        fire_rows(0, 0)
        fire_idx(1, 1)
        step(0, 0, True, True)
        step(1, 1, True, True)

        def jbody(j, _):
            step(2 * j, 0, True, True)
            step(2 * j + 1, 1, True, True)
            return 0

        lax.fori_loop(1, CH // 2 - 1, jbody, 0)
        step(CH - 2, 0, True, False)
        step(CH - 1, 1, False, False)

        plsc.subcore_barrier()
        for t in range(RPT // GE):
            r = sid * RPT + t * GE
            pltpu.sync_copy(out_s.at[pl.ds(r, GE)], xws_v)
            pltpu.sync_copy(xws_v, out_hbm.at[cid, pl.ds(r, GE)])
            pltpu.sync_copy(den_s.at[pl.ds(r, GE)], ste_v)
            pltpu.sync_copy(ste_v, den_hbm.at[cid, pl.ds(r, GE)])

    return edge_pass


_sc_edge_h8 = _make_sc_edge(8)
_sc_edge_h1 = _make_sc_edge(1)


@functools.partial(
    pl.kernel,
    out_type=jax.ShapeDtypeStruct((B, HID), _f32),
    mesh=_MESH,
    compiler_params=_SC_PARAMS,
    scratch_types=(
        pltpu.VMEM((B // NW,), _i32),
        pltpu.VMEM((B // NW, HID), _f32),
        pltpu.SemaphoreType.DMA,
    ),
)
def _sc_gather_rows(ids_hbm, x_hbm, out_hbm, idx_v, rows_v, sem):
    _, _, wid = _tile_ids()
    base = wid * (B // NW)
    pltpu.sync_copy(ids_hbm.at[pl.ds(base, B // NW)], idx_v)
    pltpu.async_copy(x_hbm.at[idx_v], rows_v, sem).wait()
    pltpu.sync_copy(rows_v, out_hbm.at[pl.ds(base, B // NW)])


_BN_INV = 0.9999950000374997


def _head_sel(chan):
    jr = lax.broadcasted_iota(_i32, (HID, 16), 0)
    hc = lax.broadcasted_iota(_i32, (HID, 16), 1)
    return (jr // chan == hc).astype(_f32)


def _head_expand(chan):
    hc = lax.broadcasted_iota(_i32, (16, HID), 0)
    jr = lax.broadcasted_iota(_i32, (16, HID), 1)
    return (jr // chan == hc).astype(_f32)


def _proj_body(x, w_ref, asf_ref, adf_ref, chan, xwe_ref, ad_ref):
    xw = jnp.dot(x, w_ref[...], preferred_element_type=_f32)
    sel = _head_sel(chan)
    xwe_ref[:, 0:HID] = xw
    xwe_ref[:, HID:XWE] = jnp.dot(xw * asf_ref[...], sel,
                                  preferred_element_type=_f32)
    ad_ref[...] = jnp.dot(xw * adf_ref[...], sel, preferred_element_type=_f32)


def _tc_proj_first(chan):
    def body(x_ref, w_ref, asf_ref, adf_ref, xwe_ref, ad_ref):
        _proj_body(x_ref[...], w_ref, asf_ref, adf_ref, chan, xwe_ref, ad_ref)
    return body


def _tc_proj_next(chan_prev, chan, relu):
    def body(p0_ref, p1_ref, d0_ref, d1_ref, b_ref, w_ref, asf_ref, adf_ref,
             xwe_ref, ad_ref):
        dinv = 1.0 / (d0_ref[...] + d1_ref[...] + 1e-16)
        scale = jnp.dot(dinv, _head_expand(chan_prev),
                        preferred_element_type=_f32)
        x = (p0_ref[...] + p1_ref[...]) * scale + b_ref[...]
        if relu:
            x = jnp.maximum(x, 0.0)
        _proj_body(x, w_ref, asf_ref, adf_ref, chan, xwe_ref, ad_ref)
    return body


_PROJ_OUT = (
    jax.ShapeDtypeStruct((NR, XWE), _f32),
    jax.ShapeDtypeStruct((NR, 16), _f32),
)
_PROJ_OUT_SPECS = (
    pl.BlockSpec((1024, XWE), lambda i: (i, 0)),
    pl.BlockSpec((1024, 16), lambda i: (i, 0)),
)
_W_SPEC = pl.BlockSpec((HID, HID), lambda i: (0, 0))
_ROW_SPEC = pl.BlockSpec((1, HID), lambda i: (0, 0))
_X_SPEC = pl.BlockSpec((1024, HID), lambda i: (i, 0))
_D_SPEC = pl.BlockSpec((1024, 16), lambda i: (i, 0))


def _run_proj_first(x_pad, w, asf, adf, chan):
    return pl.pallas_call(
        _tc_proj_first(chan),
        grid=(NR // 1024,),
        in_specs=[_X_SPEC, _W_SPEC, _ROW_SPEC, _ROW_SPEC],
        out_shape=_PROJ_OUT,
        out_specs=_PROJ_OUT_SPECS,
    )(x_pad, w, asf, adf)


def _run_proj_next(out_p, den_p, bias, w, asf, adf, chan_prev, chan):
    return pl.pallas_call(
        _tc_proj_next(chan_prev, chan, True),
        grid=(NR // 1024,),
        in_specs=[_X_SPEC, _X_SPEC, _D_SPEC, _D_SPEC, _ROW_SPEC,
                  _W_SPEC, _ROW_SPEC, _ROW_SPEC],
        out_shape=_PROJ_OUT,
        out_specs=_PROJ_OUT_SPECS,
    )(out_p[0], out_p[1], den_p[0], den_p[1], bias, w, asf, adf)


def _loop_attr_body(ea0_ref, ea1_ref, c0_ref, c1_ref, out_ref):
    cnt = jnp.maximum(c0_ref[...] + c1_ref[...], 1.0)
    out_ref[...] = (ea0_ref[...] + ea1_ref[...]) / cnt


def _edge_attn_body(ea_ref, we1_ref, af1_ref, we2_ref, af2_ref,
                    we3_ref, af3_ref, o1_ref, o2_ref, o3_ref):
    ea = ea_ref[...]
    s16 = _head_sel(16)
    s1 = _head_sel(HID)
    for we, af, sel, out in ((we1_ref, af1_ref, s16, o1_ref),
                             (we2_ref, af2_ref, s16, o2_ref),
                             (we3_ref, af3_ref, s1, o3_ref)):
        m = jnp.dot(we[...] * af[...], sel, preferred_element_type=_f32)
        out[...] = jnp.dot(ea, m, preferred_element_type=_f32)


def _final_x_body(p0_ref, p1_ref, d0_ref, d1_ref, b_ref, out_ref):
    dinv = 1.0 / (d0_ref[...] + d1_ref[...] + 1e-16)
    scale = jnp.dot(dinv, _head_expand(HID), preferred_element_type=_f32)
    out_ref[...] = (p0_ref[...] + p1_ref[...]) * scale + b_ref[...]


def _bn(x, g, b):
    return x * (g * _BN_INV) + b


def _fuse_body(nf_ref, w1_ref, b1_ref, g1_ref, be1_ref,
               w2_ref, b2_ref, g2_ref, be2_ref,
               ne_ref, fw_ref, fb_ref, fg_ref, fbe_ref, out_ref):
    h = jnp.dot(nf_ref[...], w1_ref[...], preferred_element_type=_f32)
    h = jnp.maximum(_bn(h + b1_ref[...], g1_ref[...], be1_ref[...]), 0.0)
    h = jnp.dot(h, w2_ref[...], preferred_element_type=_f32)
    h = jnp.maximum(_bn(h + b2_ref[...], g2_ref[...], be2_ref[...]), 0.0)
    f = (jnp.dot(ne_ref[...], fw_ref[0:HID, :], preferred_element_type=_f32)
         + jnp.dot(h, fw_ref[HID:2 * HID, :], preferred_element_type=_f32))
    f = jnp.maximum(_bn(f + fb_ref[...], fg_ref[...], fbe_ref[...]), 0.0)
    out_ref[...] = f


def _cls_body(f_ref, w_ref, b_ref, out_ref):
    out_ref[...] = (jnp.dot(f_ref[...], w_ref[...],
                            preferred_element_type=_f32) + b_ref[...])


def kernel(current_node_ids, network_features, edge_index, edge_attr, params):
    src = edge_index[0]
    dst = edge_index[1]
    loop_ids = jnp.arange(N, dtype=_i32)
    pad_a = N + jnp.arange(EN_PAD - E - N, dtype=_i32) % (NR - N)
    pad_0 = N + jnp.arange(E_PAD - E, dtype=_i32) % (NR - N)
    s2 = jnp.concatenate([src, loop_ids, pad_a])
    d2 = jnp.concatenate([dst, loop_ids, pad_a])
    d0 = jnp.concatenate([dst, pad_0])
    ea_pad = jnp.pad(edge_attr, ((0, E_PAD - E), (0, 0)))

    easum_p, cnt_p = _sc_loopattr(d0, ea_pad)
    loop_attr = pl.pallas_call(
        _loop_attr_body,
        grid=(NR // 1024,),
        in_specs=[_D_SPEC] * 4,
        out_shape=jax.ShapeDtypeStruct((NR, EDIM), _f32),
        out_specs=_D_SPEC,
    )(easum_p[0], easum_p[1], cnt_p[0], cnt_p[1])

    ea2 = jnp.concatenate(
        [edge_attr, loop_attr[:N], jnp.zeros((EN_PAD - E - N, EDIM), _f32)])

    g1p, g2p, g3p = params['gat1'], params['gat2'], params['gat3']
    af = [p['att_e'].reshape(1, HID) for p in (g1p, g2p, g3p)]
    ae1, ae2, ae3 = pl.pallas_call(
        _edge_attn_body,
        grid=(EN_PAD // 2048,),
        in_specs=[
            pl.BlockSpec((2048, EDIM), lambda i: (i, 0)),
            pl.BlockSpec((EDIM, HID), lambda i: (0, 0)),
            _ROW_SPEC,
            pl.BlockSpec((EDIM, HID), lambda i: (0, 0)),
            _ROW_SPEC,
            pl.BlockSpec((EDIM, HID), lambda i: (0, 0)),
            _ROW_SPEC,
        ],
        out_shape=tuple(
            jax.ShapeDtypeStruct((EN_PAD, EDIM), _f32) for _ in range(3)),
        out_specs=tuple(
            pl.BlockSpec((2048, EDIM), lambda i: (i, 0)) for _ in range(3)),
    )(ea2, g1p['W_e'], af[0], g2p['W_e'], af[1], g3p['W_e'], af[2])

    emb_pad = jnp.pad(params['emb'], ((0, NR - N), (0, 0)))

    xwe, adst_t = _run_proj_first(
        emb_pad, g1p['W'], g1p['att_src'].reshape(1, HID),
        g1p['att_dst'].reshape(1, HID), 16)
    out_p, den_p = _sc_edge_h8(s2, d2, adst_t, ae1, xwe)

    xwe, adst_t = _run_proj_next(
        out_p, den_p, g1p['b'].reshape(1, HID), g2p['W'],
        g2p['att_src'].reshape(1, HID), g2p['att_dst'].reshape(1, HID),
        16, 16)
    out_p, den_p = _sc_edge_h8(s2, d2, adst_t, ae2, xwe)

    xwe, adst_t = _run_proj_next(
        out_p, den_p, g2p['b'].reshape(1, HID), g3p['W'],
        g3p['att_src'].reshape(1, HID), g3p['att_dst'].reshape(1, HID),
        16, HID)
    out_p, den_p = _sc_edge_h1(s2, d2, adst_t, ae3, xwe)

    x3 = pl.pallas_call(
        _final_x_body,
        grid=(NR // 1024,),
        in_specs=[_X_SPEC, _X_SPEC, _D_SPEC, _D_SPEC, _ROW_SPEC],
        out_shape=jax.ShapeDtypeStruct((NR, HID), _f32),
        out_specs=_X_SPEC,
    )(out_p[0], out_p[1], den_p[0], den_p[1], g3p['b'].reshape(1, HID))

    node_emb = _sc_gather_rows(current_node_ids, x3)

    fused = pl.pallas_call(
        _fuse_body,
        out_shape=jax.ShapeDtypeStruct((B, HID), _f32),
    )(network_features,
      params['ne_W1'], params['ne_b1'].reshape(1, HID),
      params['ne_g1'].reshape(1, HID), params['ne_be1'].reshape(1, HID),
      params['ne_W2'], params['ne_b2'].reshape(1, HID),
      params['ne_g2'].reshape(1, HID), params['ne_be2'].reshape(1, HID),
      node_emb, params['fus_W'], params['fus_b'].reshape(1, HID),
      params['fus_g'].reshape(1, HID), params['fus_be'].reshape(1, HID))

    cls_w = jnp.pad(params['cls_W'], ((0, 0), (0, NR - N)))
    cls_b = jnp.pad(params['cls_b'], (0, NR - N)).reshape(1, NR)
    logits = pl.pallas_call(
        _cls_body,
        grid=(NR // 1024,),
        in_specs=[
            pl.BlockSpec((B, HID), lambda i: (0, 0)),
            pl.BlockSpec((HID, 1024), lambda i: (0, i)),
            pl.BlockSpec((1, 1024), lambda i: (0, i)),
        ],
        out_shape=jax.ShapeDtypeStruct((B, NR), _f32),
        out_specs=pl.BlockSpec((B, 1024), lambda i: (0, i)),
    )(fused, cls_w, cls_b)
    return logits[:, :N]


# speedup vs baseline: 31.6736x; 1.0455x over previous
import functools

import jax
import jax.numpy as jnp
from jax import lax
from jax.experimental import pallas as pl
from jax.experimental.pallas import tpu as pltpu
from jax.experimental.pallas import tpu_sc as plsc

N = 10000
E = 320000
HID = 128
EDIM = 16
NF = 16
B = 1024

G = 128
NW = 32
NR = 10240
RPT = NR // 16
PAD_ROW = NR - 1

CH_A = 82
EN_PAD = NW * CH_A * G
PT_A = CH_A * G
XWE = HID + 16

CH_0 = 79
E_PAD = NW * CH_0 * G
PT_0 = CH_0 * G

_f32 = jnp.float32
_i32 = jnp.int32

_MESH = plsc.VectorSubcoreMesh(core_axis_name="c", subcore_axis_name="s")
_SC_PARAMS = pltpu.CompilerParams(use_tc_tiling_on_sc=False,
                                  needs_layout_passes=False)


def _tile_ids():
    cid = lax.axis_index("c")
    sid = lax.axis_index("s")
    return cid, sid, cid * 16 + sid


def _fill_rows(ref, nrows, ncols, val):
    def body(i, _):
        for h in range(ncols // 16):
            ref[i, pl.ds(16 * h, 16)] = jnp.full((16,), val, _f32)
        return 0
    lax.fori_loop(0, nrows, body, 0)


@functools.partial(
    pl.kernel,
    out_type=(
        jax.ShapeDtypeStruct((2, NR, EDIM), _f32),
        jax.ShapeDtypeStruct((2, NR, EDIM), _f32),
    ),
    mesh=_MESH,
    compiler_params=_SC_PARAMS,
    scratch_types=(
        pltpu.VMEM((G,), _i32),
        pltpu.VMEM((G, EDIM), _f32),
        pltpu.VMEM((G, EDIM), _f32),
        pltpu.VMEM_SHARED((NR, EDIM), _f32),
        pltpu.VMEM_SHARED((NR, EDIM), _f32),
    ),
)
def _sc_loopattr(d_hbm, ea_hbm, easum_out, cnt_out,
                 didx_v, ear_v, ones_v, accea_s, acccnt_s):
    cid, sid, wid = _tile_ids()
    _fill_rows(ear_v, G, EDIM, 0.0)
    _fill_rows(ones_v, G, EDIM, 1.0)
    for t in range(RPT // G):
        pltpu.sync_copy(ear_v, accea_s.at[pl.ds(sid * RPT + t * G, G)])
        pltpu.sync_copy(ear_v, acccnt_s.at[pl.ds(sid * RPT + t * G, G)])
    plsc.subcore_barrier()

    def chunk(k, _):
        base = wid * PT_0 + k * G
        pltpu.sync_copy(d_hbm.at[pl.ds(base, G)], didx_v)
        pltpu.sync_copy(ea_hbm.at[pl.ds(base, G)], ear_v)
        pltpu.sync_copy(ear_v, accea_s.at[didx_v], add=True)
        pltpu.sync_copy(ones_v, acccnt_s.at[didx_v], add=True)
        return 0

    lax.fori_loop(0, CH_0, chunk, 0)
    plsc.subcore_barrier()
    for t in range(RPT // G):
        r = sid * RPT + t * G
        pltpu.sync_copy(accea_s.at[pl.ds(r, G)], ear_v)
        pltpu.sync_copy(ear_v, easum_out.at[cid, pl.ds(r, G)])
        pltpu.sync_copy(acccnt_s.at[pl.ds(r, G)], ones_v)
        pltpu.sync_copy(ones_v, cnt_out.at[cid, pl.ds(r, G)])


def _make_sc_edge(nheads):
    GE = 64
    CH = PT_A // GE

    @functools.partial(
        pl.kernel,
        out_type=(
            jax.ShapeDtypeStruct((2, NR, HID), _f32),
            jax.ShapeDtypeStruct((2, NR, EDIM), _f32),
        ),
        mesh=_MESH,
        compiler_params=_SC_PARAMS,
        scratch_types=(
            pltpu.VMEM((2, GE), _i32),
            pltpu.VMEM((2, GE), _i32),
            pltpu.VMEM((2, GE, XWE), _f32),
            pltpu.VMEM((2, GE, EDIM), _f32),
            pltpu.VMEM((2, GE, EDIM), _f32),
            pltpu.VMEM((GE, EDIM), _f32),
            pltpu.VMEM((GE, HID), _f32),
            pltpu.VMEM((GE, EDIM), _f32),
            pltpu.VMEM((GE,), _i32),
            pltpu.VMEM_SHARED((NR, HID), _f32),
            pltpu.VMEM_SHARED((NR, EDIM), _f32),
        ) + (pltpu.SemaphoreType.DMA,) * 10,
    )
    def edge_pass(s2_hbm, d2_hbm, adst_hbm, ae_hbm, xwe_hbm,
                  out_hbm, den_hbm,
                  sidx_v, didx_v, xwe_v, adr_v, aer_v, exr_v, xws_v, ste_v,
                  dsc_v, out_s, den_s, *sems):
        cid, sid, wid = _tile_ids()
        s_si = sems[0:2]
        s_di = sems[2:4]
        s_gx = sems[4:6]
        s_ga = sems[6:8]
        s_ge = sems[8:10]

        def fire_idx(k, s):
            base = wid * PT_A + k * GE
            pltpu.async_copy(s2_hbm.at[pl.ds(base, GE)], sidx_v.at[s], s_si[s])
            pltpu.async_copy(d2_hbm.at[pl.ds(base, GE)], didx_v.at[s], s_di[s])

        def wait_idx(s):
            pltpu.make_async_copy(
                s2_hbm.at[pl.ds(0, GE)], sidx_v.at[s], s_si[s]).wait()
            pltpu.make_async_copy(
                d2_hbm.at[pl.ds(0, GE)], didx_v.at[s], s_di[s]).wait()

        def fire_rows(k, s):
            base = wid * PT_A + k * GE
            pltpu.async_copy(xwe_hbm.at[sidx_v.at[s]], xwe_v.at[s], s_gx[s])
            pltpu.async_copy(adst_hbm.at[didx_v.at[s]], adr_v.at[s], s_ga[s])
            pltpu.async_copy(ae_hbm.at[pl.ds(base, GE)], aer_v.at[s], s_ge[s])

        def wait_rows(s):
            pltpu.make_async_copy(
                xwe_hbm.at[sidx_v.at[s]], xwe_v.at[s], s_gx[s]).wait()
            pltpu.make_async_copy(
                adst_hbm.at[didx_v.at[s]], adr_v.at[s], s_ga[s]).wait()
            pltpu.make_async_copy(
                ae_hbm.at[pl.ds(0, GE)], aer_v.at[s], s_ge[s]).wait()

        def compute(s):
            dnums = lax.GatherDimensionNumbers(
                offset_dims=(), collapsed_slice_dims=(0,),
                start_index_map=(0,))

            def row(i, _):
                a = (xwe_v[s, i, pl.ds(HID, 16)] + adr_v[s, i]
                     + aer_v[s, i])
                a = jnp.where(a > 0.0, a, 0.2 * a)
                e = jnp.exp(a)
                exr_v[i] = e
                for h in range(HID // 16):
                    hh = h if nheads == 8 else 0
                    m = lax.gather(
                        e, jnp.full((16, 1), hh, _i32), dnums, (1,),
                        mode=lax.GatherScatterMode.PROMISE_IN_BOUNDS)
                    xws_v[i, pl.ds(16 * h, 16)] = (
                        xwe_v[s, i, pl.ds(16 * h, 16)] * m)
                return 0

            lax.fori_loop(0, GE, row, 0)

        def step(k, s, do_rows_next, do_idx2):
            if do_rows_next:
                wait_idx(1 - s)
            wait_rows(s)
            if do_rows_next:
                fire_rows(k + 1, 1 - s)
            for t in range(GE // 16):
                dsc_v[pl.ds(16 * t, 16)] = didx_v[s, pl.ds(16 * t, 16)]
            if do_idx2:
                fire_idx(k + 2, s)
            compute(s)
            pltpu.sync_copy(xws_v, out_s.at[dsc_v], add=True)
            pltpu.sync_copy(exr_v, den_s.at[dsc_v], add=True)

        _fill_rows(xws_v, GE, HID, 0.0)
        _fill_rows(ste_v, GE, EDIM, 0.0)
        for t in range(RPT // GE):
            pltpu.sync_copy(xws_v, out_s.at[pl.ds(sid * RPT + t * GE, GE)])
            pltpu.sync_copy(ste_v, den_s.at[pl.ds(sid * RPT + t * GE, GE)])
        plsc.subcore_barrier()

        fire_idx(0, 0)
        wait_idx(0)
        fire_rows(0, 0)
        fire_idx(1, 1)
        step(0, 0, True, True)
        step(1, 1, True, True)

        def jbody(j, _):
            step(2 * j, 0, True, True)
            step(2 * j + 1, 1, True, True)
            return 0

        lax.fori_loop(1, CH // 2 - 1, jbody, 0)
        step(CH - 2, 0, True, False)
        step(CH - 1, 1, False, False)

        plsc.subcore_barrier()
        for t in range(RPT // GE):
            r = sid * RPT + t * GE
            pltpu.sync_copy(out_s.at[pl.ds(r, GE)], xws_v)
            pltpu.sync_copy(xws_v, out_hbm.at[cid, pl.ds(r, GE)])
            pltpu.sync_copy(den_s.at[pl.ds(r, GE)], ste_v)
            pltpu.sync_copy(ste_v, den_hbm.at[cid, pl.ds(r, GE)])

    return edge_pass


_sc_edge_h8 = _make_sc_edge(8)
_sc_edge_h1 = _make_sc_edge(1)


@functools.partial(
    pl.kernel,
    out_type=jax.ShapeDtypeStruct((B, HID), _f32),
    mesh=_MESH,
    compiler_params=_SC_PARAMS,
    scratch_types=(
        pltpu.VMEM((B // NW,), _i32),
        pltpu.VMEM((B // NW, HID), _f32),
        pltpu.SemaphoreType.DMA,
    ),
)
def _sc_gather_rows(ids_hbm, x_hbm, out_hbm, idx_v, rows_v, sem):
    _, _, wid = _tile_ids()
    base = wid * (B // NW)
    pltpu.sync_copy(ids_hbm.at[pl.ds(base, B // NW)], idx_v)
    pltpu.async_copy(x_hbm.at[idx_v], rows_v, sem).wait()
    pltpu.sync_copy(rows_v, out_hbm.at[pl.ds(base, B // NW)])


_BN_INV = 0.9999950000374997


def _head_sel(chan):
    jr = lax.broadcasted_iota(_i32, (HID, 16), 0)
    hc = lax.broadcasted_iota(_i32, (HID, 16), 1)
    return (jr // chan == hc).astype(_f32)


def _head_expand(chan):
    hc = lax.broadcasted_iota(_i32, (16, HID), 0)
    jr = lax.broadcasted_iota(_i32, (16, HID), 1)
    return (jr // chan == hc).astype(_f32)


def _proj_body(x, w_ref, asf_ref, adf_ref, chan, xwe_ref, ad_ref):
    xw = jnp.dot(x, w_ref[...], preferred_element_type=_f32)
    sel = _head_sel(chan)
    xwe_ref[:, 0:HID] = xw
    xwe_ref[:, HID:XWE] = jnp.dot(xw * asf_ref[...], sel,
                                  preferred_element_type=_f32)
    ad_ref[...] = jnp.dot(xw * adf_ref[...], sel, preferred_element_type=_f32)


def _tc_proj_first(chan):
    def body(x_ref, w_ref, asf_ref, adf_ref, xwe_ref, ad_ref):
        _proj_body(x_ref[...], w_ref, asf_ref, adf_ref, chan, xwe_ref, ad_ref)
    return body


def _tc_proj_next(chan_prev, chan, relu):
    def body(p0_ref, p1_ref, d0_ref, d1_ref, b_ref, w_ref, asf_ref, adf_ref,
             xwe_ref, ad_ref):
        dinv = 1.0 / (d0_ref[...] + d1_ref[...] + 1e-16)
        scale = jnp.dot(dinv, _head_expand(chan_prev),
                        preferred_element_type=_f32)
        x = (p0_ref[...] + p1_ref[...]) * scale + b_ref[...]
        if relu:
            x = jnp.maximum(x, 0.0)
        _proj_body(x, w_ref, asf_ref, adf_ref, chan, xwe_ref, ad_ref)
    return body


_PROJ_OUT = (
    jax.ShapeDtypeStruct((NR, XWE), _f32),
    jax.ShapeDtypeStruct((NR, 16), _f32),
)
_PROJ_OUT_SPECS = (
    pl.BlockSpec((1024, XWE), lambda i: (i, 0)),
    pl.BlockSpec((1024, 16), lambda i: (i, 0)),
)
_W_SPEC = pl.BlockSpec((HID, HID), lambda i: (0, 0))
_ROW_SPEC = pl.BlockSpec((1, HID), lambda i: (0, 0))
_X_SPEC = pl.BlockSpec((1024, HID), lambda i: (i, 0))
_D_SPEC = pl.BlockSpec((1024, 16), lambda i: (i, 0))


def _run_proj_first(x_pad, w, asf, adf, chan):
    return pl.pallas_call(
        _tc_proj_first(chan),
        grid=(NR // 1024,),
        in_specs=[_X_SPEC, _W_SPEC, _ROW_SPEC, _ROW_SPEC],
        out_shape=_PROJ_OUT,
        out_specs=_PROJ_OUT_SPECS,
    )(x_pad, w, asf, adf)


def _run_proj_next(out_p, den_p, bias, w, asf, adf, chan_prev, chan):
    return pl.pallas_call(
        _tc_proj_next(chan_prev, chan, True),
        grid=(NR // 1024,),
        in_specs=[_X_SPEC, _X_SPEC, _D_SPEC, _D_SPEC, _ROW_SPEC,
                  _W_SPEC, _ROW_SPEC, _ROW_SPEC],
        out_shape=_PROJ_OUT,
        out_specs=_PROJ_OUT_SPECS,
    )(out_p[0], out_p[1], den_p[0], den_p[1], bias, w, asf, adf)


def _loop_attr_body(ea0_ref, ea1_ref, c0_ref, c1_ref, out_ref):
    cnt = jnp.maximum(c0_ref[...] + c1_ref[...], 1.0)
    out_ref[...] = (ea0_ref[...] + ea1_ref[...]) / cnt


def _edge_attn_body(ea_ref, we1_ref, af1_ref, we2_ref, af2_ref,
                    we3_ref, af3_ref, o1_ref, o2_ref, o3_ref):
    ea = ea_ref[...]
    s16 = _head_sel(16)
    s1 = _head_sel(HID)
    for we, af, sel, out in ((we1_ref, af1_ref, s16, o1_ref),
                             (we2_ref, af2_ref, s16, o2_ref),
                             (we3_ref, af3_ref, s1, o3_ref)):
        m = jnp.dot(we[...] * af[...], sel, preferred_element_type=_f32)
        out[...] = jnp.dot(ea, m, preferred_element_type=_f32)


def _final_x_body(p0_ref, p1_ref, d0_ref, d1_ref, b_ref, out_ref):
    dinv = 1.0 / (d0_ref[...] + d1_ref[...] + 1e-16)
    scale = jnp.dot(dinv, _head_expand(HID), preferred_element_type=_f32)
    out_ref[...] = (p0_ref[...] + p1_ref[...]) * scale + b_ref[...]


def _bn(x, g, b):
    return x * (g * _BN_INV) + b


def _fuse_body(nf_ref, w1_ref, b1_ref, g1_ref, be1_ref,
               w2_ref, b2_ref, g2_ref, be2_ref,
               ne_ref, fw_ref, fb_ref, fg_ref, fbe_ref, out_ref):
    h = jnp.dot(nf_ref[...], w1_ref[...], preferred_element_type=_f32)
    h = jnp.maximum(_bn(h + b1_ref[...], g1_ref[...], be1_ref[...]), 0.0)
    h = jnp.dot(h, w2_ref[...], preferred_element_type=_f32)
    h = jnp.maximum(_bn(h + b2_ref[...], g2_ref[...], be2_ref[...]), 0.0)
    f = (jnp.dot(ne_ref[...], fw_ref[0:HID, :], preferred_element_type=_f32)
         + jnp.dot(h, fw_ref[HID:2 * HID, :], preferred_element_type=_f32))
    f = jnp.maximum(_bn(f + fb_ref[...], fg_ref[...], fbe_ref[...]), 0.0)
    out_ref[...] = f


def _cls_body(f_ref, w_ref, b_ref, out_ref):
    out_ref[...] = (jnp.dot(f_ref[...], w_ref[...],
                            preferred_element_type=_f32) + b_ref[...])


def kernel(current_node_ids, network_features, edge_index, edge_attr, params):
    src = edge_index[0]
    dst = edge_index[1]
    loop_ids = jnp.arange(N, dtype=_i32)
    pad_a = N + jnp.arange(EN_PAD - E - N, dtype=_i32) % (NR - N)
    pad_0 = N + jnp.arange(E_PAD - E, dtype=_i32) % (NR - N)
    s2 = jnp.concatenate([src, loop_ids, pad_a])
    d2 = jnp.concatenate([dst, loop_ids, pad_a])
    d0 = jnp.concatenate([dst, pad_0])
    ea_pad = jnp.pad(edge_attr, ((0, E_PAD - E), (0, 0)))

    easum_p, cnt_p = _sc_loopattr(d0, ea_pad)
    loop_attr = pl.pallas_call(
        _loop_attr_body,
        grid=(NR // 1024,),
        in_specs=[_D_SPEC] * 4,
        out_shape=jax.ShapeDtypeStruct((NR, EDIM), _f32),
        out_specs=_D_SPEC,
    )(easum_p[0], easum_p[1], cnt_p[0], cnt_p[1])

    ea2 = jnp.concatenate(
        [edge_attr, loop_attr[:N], jnp.zeros((EN_PAD - E - N, EDIM), _f32)])

    g1p, g2p, g3p = params['gat1'], params['gat2'], params['gat3']
    af = [p['att_e'].reshape(1, HID) for p in (g1p, g2p, g3p)]
    ae1, ae2, ae3 = pl.pallas_call(
        _edge_attn_body,
        grid=(EN_PAD // 2048,),
        in_specs=[
            pl.BlockSpec((2048, EDIM), lambda i: (i, 0)),
            pl.BlockSpec((EDIM, HID), lambda i: (0, 0)),
            _ROW_SPEC,
            pl.BlockSpec((EDIM, HID), lambda i: (0, 0)),
            _ROW_SPEC,
            pl.BlockSpec((EDIM, HID), lambda i: (0, 0)),
            _ROW_SPEC,
        ],
        out_shape=tuple(
            jax.ShapeDtypeStruct((EN_PAD, EDIM), _f32) for _ in range(3)),
        out_specs=tuple(
            pl.BlockSpec((2048, EDIM), lambda i: (i, 0)) for _ in range(3)),
    )(ea2, g1p['W_e'], af[0], g2p['W_e'], af[1], g3p['W_e'], af[2])

    emb_pad = jnp.pad(params['emb'], ((0, NR - N), (0, 0)))

    xwe, adst_t = _run_proj_first(
        emb_pad, g1p['W'], g1p['att_src'].reshape(1, HID),
        g1p['att_dst'].reshape(1, HID), 16)
    out_p, den_p = _sc_edge_h8(s2, d2, adst_t, ae1, xwe)

    xwe, adst_t = _run_proj_next(
        out_p, den_p, g1p['b'].reshape(1, HID), g2p['W'],
        g2p['att_src'].reshape(1, HID), g2p['att_dst'].reshape(1, HID),
        16, 16)
    out_p, den_p = _sc_edge_h8(s2, d2, adst_t, ae2, xwe)

    xwe, adst_t = _run_proj_next(
        out_p, den_p, g2p['b'].reshape(1, HID), g3p['W'],
        g3p['att_src'].reshape(1, HID), g3p['att_dst'].reshape(1, HID),
        16, HID)
    out_p, den_p = _sc_edge_h1(s2, d2, adst_t, ae3, xwe)

    x3 = pl.pallas_call(
        _final_x_body,
        grid=(NR // 1024,),
        in_specs=[_X_SPEC, _X_SPEC, _D_SPEC, _D_SPEC, _ROW_SPEC],
        out_shape=jax.ShapeDtypeStruct((NR, HID), _f32),
        out_specs=_X_SPEC,
    )(out_p[0], out_p[1], den_p[0], den_p[1], g3p['b'].reshape(1, HID))

    node_emb = _sc_gather_rows(current_node_ids, x3)

    fused = pl.pallas_call(
        _fuse_body,
        out_shape=jax.ShapeDtypeStruct((B, HID), _f32),
    )(network_features,
      params['ne_W1'], params['ne_b1'].reshape(1, HID),
      params['ne_g1'].reshape(1, HID), params['ne_be1'].reshape(1, HID),
      params['ne_W2'], params['ne_b2'].reshape(1, HID),
      params['ne_g2'].reshape(1, HID), params['ne_be2'].reshape(1, HID),
      node_emb, params['fus_W'], params['fus_b'].reshape(1, HID),
      params['fus_g'].reshape(1, HID), params['fus_be'].reshape(1, HID))

    cls_w = jnp.pad(params['cls_W'], ((0, 0), (0, NR - N)))
    cls_b = jnp.pad(params['cls_b'], (0, NR - N)).reshape(1, NR)
    logits = pl.pallas_call(
        _cls_body,
        grid=(NR // 1024,),
        in_specs=[
            pl.BlockSpec((B, HID), lambda i: (0, 0)),
            pl.BlockSpec((HID, 1024), lambda i: (0, i)),
            pl.BlockSpec((1, 1024), lambda i: (0, i)),
        ],
        out_shape=jax.ShapeDtypeStruct((B, NR), _f32),
        out_specs=pl.BlockSpec((B, 1024), lambda i: (0, i)),
    )(fused, cls_w, cls_b)
    return logits[:, :N]


# speedup vs baseline: 53.1453x; 1.6779x over previous
import functools

import jax
import jax.numpy as jnp
from jax import lax
from jax.experimental import pallas as pl
from jax.experimental.pallas import tpu as pltpu
from jax.experimental.pallas import tpu_sc as plsc

N = 10000
E = 320000
HID = 128
EDIM = 16
NF = 16
B = 1024

G = 128
NW = 32
NR = 10240
RPT = NR // 16
PAD_ROW = NR - 1

CH_A = 82
EN_PAD = NW * CH_A * G
PT_A = CH_A * G
XWE = HID + 16

CH_0 = 79
E_PAD = NW * CH_0 * G
PT_0 = CH_0 * G

_f32 = jnp.float32
_i32 = jnp.int32

_MESH = plsc.VectorSubcoreMesh(core_axis_name="c", subcore_axis_name="s")
_SC_PARAMS = pltpu.CompilerParams(use_tc_tiling_on_sc=False,
                                  needs_layout_passes=False)


def _tile_ids():
    cid = lax.axis_index("c")
    sid = lax.axis_index("s")
    return cid, sid, cid * 16 + sid


def _fill_rows(ref, nrows, ncols, val):
    def body(i, _):
        for h in range(ncols // 16):
            ref[i, pl.ds(16 * h, 16)] = jnp.full((16,), val, _f32)
        return 0
    lax.fori_loop(0, nrows, body, 0)


@functools.partial(
    pl.kernel,
    out_type=(
        jax.ShapeDtypeStruct((2, NR, EDIM), _f32),
        jax.ShapeDtypeStruct((2, NR, EDIM), _f32),
    ),
    mesh=_MESH,
    compiler_params=_SC_PARAMS,
    scratch_types=(
        pltpu.VMEM((G,), _i32),
        pltpu.VMEM((G, EDIM), _f32),
        pltpu.VMEM((G, EDIM), _f32),
        pltpu.VMEM_SHARED((NR, EDIM), _f32),
        pltpu.VMEM_SHARED((NR, EDIM), _f32),
    ),
)
def _sc_loopattr(d_hbm, ea_hbm, easum_out, cnt_out,
                 didx_v, ear_v, ones_v, accea_s, acccnt_s):
    cid, sid, wid = _tile_ids()
    _fill_rows(ear_v, G, EDIM, 0.0)
    _fill_rows(ones_v, G, EDIM, 1.0)
    for t in range(RPT // G):
        pltpu.sync_copy(ear_v, accea_s.at[pl.ds(sid * RPT + t * G, G)])
        pltpu.sync_copy(ear_v, acccnt_s.at[pl.ds(sid * RPT + t * G, G)])
    plsc.subcore_barrier()

    def chunk(k, _):
        base = wid * PT_0 + k * G
        pltpu.sync_copy(d_hbm.at[pl.ds(base, G)], didx_v)
        pltpu.sync_copy(ea_hbm.at[pl.ds(base, G)], ear_v)
        pltpu.sync_copy(ear_v, accea_s.at[didx_v], add=True)
        pltpu.sync_copy(ones_v, acccnt_s.at[didx_v], add=True)
        return 0

    lax.fori_loop(0, CH_0, chunk, 0)
    plsc.subcore_barrier()
    for t in range(RPT // G):
        r = sid * RPT + t * G
        pltpu.sync_copy(accea_s.at[pl.ds(r, G)], ear_v)
        pltpu.sync_copy(ear_v, easum_out.at[cid, pl.ds(r, G)])
        pltpu.sync_copy(acccnt_s.at[pl.ds(r, G)], ones_v)
        pltpu.sync_copy(ones_v, cnt_out.at[cid, pl.ds(r, G)])


def _make_sc_edge(nheads):
    GE = 64
    CH = PT_A // GE

    @functools.partial(
        pl.kernel,
        out_type=(
            jax.ShapeDtypeStruct((2, NR, HID), _f32),
            jax.ShapeDtypeStruct((2, NR, EDIM), _f32),
        ),
        mesh=_MESH,
        compiler_params=_SC_PARAMS,
        scratch_types=(
            pltpu.VMEM((2, GE), _i32),
            pltpu.VMEM((2, GE), _i32),
            pltpu.VMEM((2, GE, XWE), _f32),
            pltpu.VMEM((2, GE, EDIM), _f32),
            pltpu.VMEM((2, GE, EDIM), _f32),
            pltpu.VMEM((GE, EDIM), _f32),
            pltpu.VMEM((GE, HID), _f32),
            pltpu.VMEM((GE, EDIM), _f32),
            pltpu.VMEM((GE,), _i32),
            pltpu.VMEM_SHARED((NR, HID), _f32),
            pltpu.VMEM_SHARED((NR, EDIM), _f32),
        ) + (pltpu.SemaphoreType.DMA,) * 10,
    )
    def edge_pass(s2_hbm, d2_hbm, adst_hbm, ae_hbm, xwe_hbm,
                  out_hbm, den_hbm,
                  sidx_v, didx_v, xwe_v, adr_v, aer_v, exr_v, xws_v, ste_v,
                  dsc_v, out_s, den_s, *sems):
        cid, sid, wid = _tile_ids()
        s_si = sems[0:2]
        s_di = sems[2:4]
        s_gx = sems[4:6]
        s_ga = sems[6:8]
        s_ge = sems[8:10]

        def fire_idx(k, s):
            base = wid * PT_A + k * GE
            pltpu.async_copy(s2_hbm.at[pl.ds(base, GE)], sidx_v.at[s], s_si[s])
            pltpu.async_copy(d2_hbm.at[pl.ds(base, GE)], didx_v.at[s], s_di[s])

        def wait_idx(s):
            pltpu.make_async_copy(
                s2_hbm.at[pl.ds(0, GE)], sidx_v.at[s], s_si[s]).wait()
            pltpu.make_async_copy(
                d2_hbm.at[pl.ds(0, GE)], didx_v.at[s], s_di[s]).wait()

        def fire_rows(k, s):
            base = wid * PT_A + k * GE
            pltpu.async_copy(xwe_hbm.at[sidx_v.at[s]], xwe_v.at[s], s_gx[s])
            pltpu.async_copy(adst_hbm.at[didx_v.at[s]], adr_v.at[s], s_ga[s])
            pltpu.async_copy(ae_hbm.at[pl.ds(base, GE)], aer_v.at[s], s_ge[s])

        def wait_rows(s):
            pltpu.make_async_copy(
                xwe_hbm.at[sidx_v.at[s]], xwe_v.at[s], s_gx[s]).wait()
            pltpu.make_async_copy(
                adst_hbm.at[didx_v.at[s]], adr_v.at[s], s_ga[s]).wait()
            pltpu.make_async_copy(
                ae_hbm.at[pl.ds(0, GE)], aer_v.at[s], s_ge[s]).wait()

        def compute(s):
            dnums = lax.GatherDimensionNumbers(
                offset_dims=(), collapsed_slice_dims=(0,),
                start_index_map=(0,))

            @functools.partial(plsc.parallel_loop, 0, GE, unroll=4)
            def row(i):
                a = (xwe_v[s, i, pl.ds(HID, 16)] + adr_v[s, i]
                     + aer_v[s, i])
                a = jnp.where(a > 0.0, a, 0.2 * a)
                e = jnp.exp(a)
                exr_v[i] = e
                for h in range(HID // 16):
                    hh = h if nheads == 8 else 0
                    m = lax.gather(
                        e, jnp.full((16, 1), hh, _i32), dnums, (1,),
                        mode=lax.GatherScatterMode.PROMISE_IN_BOUNDS)
                    xws_v[i, pl.ds(16 * h, 16)] = (
                        xwe_v[s, i, pl.ds(16 * h, 16)] * m)

        def step(k, s, do_rows_next, do_idx2):
            if do_rows_next:
                wait_idx(1 - s)
            wait_rows(s)
            if do_rows_next:
                fire_rows(k + 1, 1 - s)
            for t in range(GE // 16):
                dsc_v[pl.ds(16 * t, 16)] = didx_v[s, pl.ds(16 * t, 16)]
            if do_idx2:
                fire_idx(k + 2, s)
            compute(s)
            pltpu.sync_copy(xws_v, out_s.at[dsc_v], add=True)
            pltpu.sync_copy(exr_v, den_s.at[dsc_v], add=True)

        _fill_rows(xws_v, GE, HID, 0.0)
        _fill_rows(ste_v, GE, EDIM, 0.0)
        for t in range(RPT // GE):
            pltpu.sync_copy(xws_v, out_s.at[pl.ds(sid * RPT + t * GE, GE)])
            pltpu.sync_copy(ste_v, den_s.at[pl.ds(sid * RPT + t * GE, GE)])
        plsc.subcore_barrier()

        fire_idx(0, 0)
        wait_idx(0)
        fire_rows(0, 0)
        fire_idx(1, 1)
        step(0, 0, True, True)
        step(1, 1, True, True)

        def jbody(j, _):
            step(2 * j, 0, True, True)
            step(2 * j + 1, 1, True, True)
            return 0

        lax.fori_loop(1, CH // 2 - 1, jbody, 0)
        step(CH - 2, 0, True, False)
        step(CH - 1, 1, False, False)

        plsc.subcore_barrier()
        for t in range(RPT // GE):
            r = sid * RPT + t * GE
            pltpu.sync_copy(out_s.at[pl.ds(r, GE)], xws_v)
            pltpu.sync_copy(xws_v, out_hbm.at[cid, pl.ds(r, GE)])
            pltpu.sync_copy(den_s.at[pl.ds(r, GE)], ste_v)
            pltpu.sync_copy(ste_v, den_hbm.at[cid, pl.ds(r, GE)])

    return edge_pass


_sc_edge_h8 = _make_sc_edge(8)
_sc_edge_h1 = _make_sc_edge(1)


@functools.partial(
    pl.kernel,
    out_type=jax.ShapeDtypeStruct((B, HID), _f32),
    mesh=_MESH,
    compiler_params=_SC_PARAMS,
    scratch_types=(
        pltpu.VMEM((B // NW,), _i32),
        pltpu.VMEM((B // NW, HID), _f32),
        pltpu.SemaphoreType.DMA,
    ),
)
def _sc_gather_rows(ids_hbm, x_hbm, out_hbm, idx_v, rows_v, sem):
    _, _, wid = _tile_ids()
    base = wid * (B // NW)
    pltpu.sync_copy(ids_hbm.at[pl.ds(base, B // NW)], idx_v)
    pltpu.async_copy(x_hbm.at[idx_v], rows_v, sem).wait()
    pltpu.sync_copy(rows_v, out_hbm.at[pl.ds(base, B // NW)])


_BN_INV = 0.9999950000374997


def _head_sel(chan):
    jr = lax.broadcasted_iota(_i32, (HID, 16), 0)
    hc = lax.broadcasted_iota(_i32, (HID, 16), 1)
    return (jr // chan == hc).astype(_f32)


def _head_expand(chan):
    hc = lax.broadcasted_iota(_i32, (16, HID), 0)
    jr = lax.broadcasted_iota(_i32, (16, HID), 1)
    return (jr // chan == hc).astype(_f32)


def _proj_body(x, w_ref, asf_ref, adf_ref, chan, xwe_ref, ad_ref):
    xw = jnp.dot(x, w_ref[...], preferred_element_type=_f32)
    sel = _head_sel(chan)
    xwe_ref[:, 0:HID] = xw
    xwe_ref[:, HID:XWE] = jnp.dot(xw * asf_ref[...], sel,
                                  preferred_element_type=_f32)
    ad_ref[...] = jnp.dot(xw * adf_ref[...], sel, preferred_element_type=_f32)


def _tc_proj_first(chan):
    def body(x_ref, w_ref, asf_ref, adf_ref, xwe_ref, ad_ref):
        _proj_body(x_ref[...], w_ref, asf_ref, adf_ref, chan, xwe_ref, ad_ref)
    return body


def _tc_proj_next(chan_prev, chan, relu):
    def body(p0_ref, p1_ref, d0_ref, d1_ref, b_ref, w_ref, asf_ref, adf_ref,
             xwe_ref, ad_ref):
        dinv = 1.0 / (d0_ref[...] + d1_ref[...] + 1e-16)
        scale = jnp.dot(dinv, _head_expand(chan_prev),
                        preferred_element_type=_f32)
        x = (p0_ref[...] + p1_ref[...]) * scale + b_ref[...]
        if relu:
            x = jnp.maximum(x, 0.0)
        _proj_body(x, w_ref, asf_ref, adf_ref, chan, xwe_ref, ad_ref)
    return body


_PROJ_OUT = (
    jax.ShapeDtypeStruct((NR, XWE), _f32),
    jax.ShapeDtypeStruct((NR, 16), _f32),
)
_PROJ_OUT_SPECS = (
    pl.BlockSpec((1024, XWE), lambda i: (i, 0)),
    pl.BlockSpec((1024, 16), lambda i: (i, 0)),
)
_W_SPEC = pl.BlockSpec((HID, HID), lambda i: (0, 0))
_ROW_SPEC = pl.BlockSpec((1, HID), lambda i: (0, 0))
_X_SPEC = pl.BlockSpec((1024, HID), lambda i: (i, 0))
_D_SPEC = pl.BlockSpec((1024, 16), lambda i: (i, 0))


def _run_proj_first(x_pad, w, asf, adf, chan):
    return pl.pallas_call(
        _tc_proj_first(chan),
        grid=(NR // 1024,),
        in_specs=[_X_SPEC, _W_SPEC, _ROW_SPEC, _ROW_SPEC],
        out_shape=_PROJ_OUT,
        out_specs=_PROJ_OUT_SPECS,
    )(x_pad, w, asf, adf)


def _run_proj_next(out_p, den_p, bias, w, asf, adf, chan_prev, chan):
    return pl.pallas_call(
        _tc_proj_next(chan_prev, chan, True),
        grid=(NR // 1024,),
        in_specs=[_X_SPEC, _X_SPEC, _D_SPEC, _D_SPEC, _ROW_SPEC,
                  _W_SPEC, _ROW_SPEC, _ROW_SPEC],
        out_shape=_PROJ_OUT,
        out_specs=_PROJ_OUT_SPECS,
    )(out_p[0], out_p[1], den_p[0], den_p[1], bias, w, asf, adf)


def _loop_attr_body(ea0_ref, ea1_ref, c0_ref, c1_ref, out_ref):
    cnt = jnp.maximum(c0_ref[...] + c1_ref[...], 1.0)
    out_ref[...] = (ea0_ref[...] + ea1_ref[...]) / cnt


def _edge_attn_body(ea_ref, we1_ref, af1_ref, we2_ref, af2_ref,
                    we3_ref, af3_ref, o1_ref, o2_ref, o3_ref):
    ea = ea_ref[...]
    s16 = _head_sel(16)
    s1 = _head_sel(HID)
    for we, af, sel, out in ((we1_ref, af1_ref, s16, o1_ref),
                             (we2_ref, af2_ref, s16, o2_ref),
                             (we3_ref, af3_ref, s1, o3_ref)):
        m = jnp.dot(we[...] * af[...], sel, preferred_element_type=_f32)
        out[...] = jnp.dot(ea, m, preferred_element_type=_f32)


def _final_x_body(p0_ref, p1_ref, d0_ref, d1_ref, b_ref, out_ref):
    dinv = 1.0 / (d0_ref[...] + d1_ref[...] + 1e-16)
    scale = jnp.dot(dinv, _head_expand(HID), preferred_element_type=_f32)
    out_ref[...] = (p0_ref[...] + p1_ref[...]) * scale + b_ref[...]


def _bn(x, g, b):
    return x * (g * _BN_INV) + b


def _fuse_body(nf_ref, w1_ref, b1_ref, g1_ref, be1_ref,
               w2_ref, b2_ref, g2_ref, be2_ref,
               ne_ref, fw_ref, fb_ref, fg_ref, fbe_ref, out_ref):
    h = jnp.dot(nf_ref[...], w1_ref[...], preferred_element_type=_f32)
    h = jnp.maximum(_bn(h + b1_ref[...], g1_ref[...], be1_ref[...]), 0.0)
    h = jnp.dot(h, w2_ref[...], preferred_element_type=_f32)
    h = jnp.maximum(_bn(h + b2_ref[...], g2_ref[...], be2_ref[...]), 0.0)
    f = (jnp.dot(ne_ref[...], fw_ref[0:HID, :], preferred_element_type=_f32)
         + jnp.dot(h, fw_ref[HID:2 * HID, :], preferred_element_type=_f32))
    f = jnp.maximum(_bn(f + fb_ref[...], fg_ref[...], fbe_ref[...]), 0.0)
    out_ref[...] = f


def _cls_body(f_ref, w_ref, b_ref, out_ref):
    out_ref[...] = (jnp.dot(f_ref[...], w_ref[...],
                            preferred_element_type=_f32) + b_ref[...])


def kernel(current_node_ids, network_features, edge_index, edge_attr, params):
    src = edge_index[0]
    dst = edge_index[1]
    loop_ids = jnp.arange(N, dtype=_i32)
    pad_a = N + jnp.arange(EN_PAD - E - N, dtype=_i32) % (NR - N)
    pad_0 = N + jnp.arange(E_PAD - E, dtype=_i32) % (NR - N)
    s2 = jnp.concatenate([src, loop_ids, pad_a])
    d2 = jnp.concatenate([dst, loop_ids, pad_a])
    d0 = jnp.concatenate([dst, pad_0])
    ea_pad = jnp.pad(edge_attr, ((0, E_PAD - E), (0, 0)))

    easum_p, cnt_p = _sc_loopattr(d0, ea_pad)
    loop_attr = pl.pallas_call(
        _loop_attr_body,
        grid=(NR // 1024,),
        in_specs=[_D_SPEC] * 4,
        out_shape=jax.ShapeDtypeStruct((NR, EDIM), _f32),
        out_specs=_D_SPEC,
    )(easum_p[0], easum_p[1], cnt_p[0], cnt_p[1])

    ea2 = jnp.concatenate(
        [edge_attr, loop_attr[:N], jnp.zeros((EN_PAD - E - N, EDIM), _f32)])

    g1p, g2p, g3p = params['gat1'], params['gat2'], params['gat3']
    af = [p['att_e'].reshape(1, HID) for p in (g1p, g2p, g3p)]
    ae1, ae2, ae3 = pl.pallas_call(
        _edge_attn_body,
        grid=(EN_PAD // 2048,),
        in_specs=[
            pl.BlockSpec((2048, EDIM), lambda i: (i, 0)),
            pl.BlockSpec((EDIM, HID), lambda i: (0, 0)),
            _ROW_SPEC,
            pl.BlockSpec((EDIM, HID), lambda i: (0, 0)),
            _ROW_SPEC,
            pl.BlockSpec((EDIM, HID), lambda i: (0, 0)),
            _ROW_SPEC,
        ],
        out_shape=tuple(
            jax.ShapeDtypeStruct((EN_PAD, EDIM), _f32) for _ in range(3)),
        out_specs=tuple(
            pl.BlockSpec((2048, EDIM), lambda i: (i, 0)) for _ in range(3)),
    )(ea2, g1p['W_e'], af[0], g2p['W_e'], af[1], g3p['W_e'], af[2])

    emb_pad = jnp.pad(params['emb'], ((0, NR - N), (0, 0)))

    xwe, adst_t = _run_proj_first(
        emb_pad, g1p['W'], g1p['att_src'].reshape(1, HID),
        g1p['att_dst'].reshape(1, HID), 16)
    out_p, den_p = _sc_edge_h8(s2, d2, adst_t, ae1, xwe)

    xwe, adst_t = _run_proj_next(
        out_p, den_p, g1p['b'].reshape(1, HID), g2p['W'],
        g2p['att_src'].reshape(1, HID), g2p['att_dst'].reshape(1, HID),
        16, 16)
    out_p, den_p = _sc_edge_h8(s2, d2, adst_t, ae2, xwe)

    xwe, adst_t = _run_proj_next(
        out_p, den_p, g2p['b'].reshape(1, HID), g3p['W'],
        g3p['att_src'].reshape(1, HID), g3p['att_dst'].reshape(1, HID),
        16, HID)
    out_p, den_p = _sc_edge_h1(s2, d2, adst_t, ae3, xwe)

    x3 = pl.pallas_call(
        _final_x_body,
        grid=(NR // 1024,),
        in_specs=[_X_SPEC, _X_SPEC, _D_SPEC, _D_SPEC, _ROW_SPEC],
        out_shape=jax.ShapeDtypeStruct((NR, HID), _f32),
        out_specs=_X_SPEC,
    )(out_p[0], out_p[1], den_p[0], den_p[1], g3p['b'].reshape(1, HID))

    node_emb = _sc_gather_rows(current_node_ids, x3)

    fused = pl.pallas_call(
        _fuse_body,
        out_shape=jax.ShapeDtypeStruct((B, HID), _f32),
    )(network_features,
      params['ne_W1'], params['ne_b1'].reshape(1, HID),
      params['ne_g1'].reshape(1, HID), params['ne_be1'].reshape(1, HID),
      params['ne_W2'], params['ne_b2'].reshape(1, HID),
      params['ne_g2'].reshape(1, HID), params['ne_be2'].reshape(1, HID),
      node_emb, params['fus_W'], params['fus_b'].reshape(1, HID),
      params['fus_g'].reshape(1, HID), params['fus_be'].reshape(1, HID))

    cls_w = jnp.pad(params['cls_W'], ((0, 0), (0, NR - N)))
    cls_b = jnp.pad(params['cls_b'], (0, NR - N)).reshape(1, NR)
    logits = pl.pallas_call(
        _cls_body,
        grid=(NR // 1024,),
        in_specs=[
            pl.BlockSpec((B, HID), lambda i: (0, 0)),
            pl.BlockSpec((HID, 1024), lambda i: (0, i)),
            pl.BlockSpec((1, 1024), lambda i: (0, i)),
        ],
        out_shape=jax.ShapeDtypeStruct((B, NR), _f32),
        out_specs=pl.BlockSpec((B, 1024), lambda i: (0, i)),
    )(fused, cls_w, cls_b)
    return logits[:, :N]
